# Initial kernel scaffold; baseline (speedup 1.0000x reference)
import jax
import jax.numpy as jnp
from jax.experimental import pallas as pl

N_NODES_K = 100000


def _linear_block(a_ref, w_ref, b_ref, o_ref, *, relu):
    a = a_ref[...]
    w = w_ref[...]
    out = jnp.dot(a, w.T, preferred_element_type=jnp.float32) + b_ref[...][None, :]
    if relu:
        out = jnp.maximum(out, 0.0)
    o_ref[...] = out


def _linear(a, w, b, relu):
    m, k = a.shape
    n = w.shape[0]
    return pl.pallas_call(
        lambda a_ref, w_ref, b_ref, o_ref: _linear_block(
            a_ref, w_ref, b_ref, o_ref, relu=relu),
        out_shape=jax.ShapeDtypeStruct((m, n), jnp.float32),
    )(a, w, b)


def kernel(x, edge_index, edge_timestamp, W1, b1, W2, b2):
    src = edge_index[0]
    dst = edge_index[1]

    msg = jnp.take(x, src, axis=0) * edge_timestamp[:, None]
    agg = jax.ops.segment_max(msg, dst, num_segments=N_NODES_K)
    agg = jnp.maximum(agg, 0.0)
    h = _linear(agg, W1, b1, relu=True)

    msg2 = jnp.take(h, src, axis=0) * edge_timestamp[:, None]
    agg2 = jax.ops.segment_max(msg2, dst, num_segments=N_NODES_K)
    agg2 = jnp.maximum(agg2, 0.0)
    out = _linear(agg2, W2, b2, relu=False)
    return out


# speedup vs baseline: 1.0033x; 1.0033x over previous
import jax
import jax.numpy as jnp
from jax.experimental import pallas as pl

N_NODES_K = 100000


def _linear_block(a_ref, w_ref, b_ref, o_ref, *, relu):
    a = a_ref[...]
    w = w_ref[...]
    out = jnp.dot(a, w.T, preferred_element_type=jnp.float32) + b_ref[...][None, :]
    if relu:
        out = jnp.maximum(out, 0.0)
    o_ref[...] = out


def _linear(a, w, b, relu):
    m, k = a.shape
    n = w.shape[0]
    bm = 4000
    return pl.pallas_call(
        lambda a_ref, w_ref, b_ref, o_ref: _linear_block(
            a_ref, w_ref, b_ref, o_ref, relu=relu),
        grid=(m // bm,),
        in_specs=[
            pl.BlockSpec((bm, k), lambda i: (i, 0)),
            pl.BlockSpec((n, k), lambda i: (0, 0)),
            pl.BlockSpec((n,), lambda i: (0,)),
        ],
        out_specs=pl.BlockSpec((bm, n), lambda i: (i, 0)),
        out_shape=jax.ShapeDtypeStruct((m, n), jnp.float32),
    )(a, w, b)


def kernel(x, edge_index, edge_timestamp, W1, b1, W2, b2):
    src = edge_index[0]
    dst = edge_index[1]

    msg = jnp.take(x, src, axis=0) * edge_timestamp[:, None]
    agg = jax.ops.segment_max(msg, dst, num_segments=N_NODES_K)
    agg = jnp.maximum(agg, 0.0)
    h = _linear(agg, W1, b1, relu=True)

    msg2 = jnp.take(h, src, axis=0) * edge_timestamp[:, None]
    agg2 = jax.ops.segment_max(msg2, dst, num_segments=N_NODES_K)
    agg2 = jnp.maximum(agg2, 0.0)
    out = _linear(agg2, W2, b2, relu=False)
    return out


# speedup vs baseline: 2.7062x; 2.6972x over previous
import dataclasses
import functools

import jax
import jax.numpy as jnp
from jax import lax
from jax.experimental import pallas as pl
from jax.experimental.pallas import tpu as pltpu
from jax.experimental.pallas import tpu_sc as plsc

N = 100_000
E = 3_200_000
F = 16
K = 1024
NB = 98
NPAD = NB * K
NC, NS = 2, 16
EC = E // NC
S = 1_802_240
G = 128
CH = 2048
GRID = NC * NB * NS
GRID_PAD = 3200
ZCH = 55

_mesh = plsc.VectorSubcoreMesh(core_axis_name="c", subcore_axis_name="s")
_cp = pltpu.CompilerParams()
if "needs_layout_passes" in pltpu.CompilerParams.__dataclass_fields__:
    _cp = dataclasses.replace(_cp, needs_layout_passes=False)
if "use_tc_tiling_on_sc" in pltpu.CompilerParams.__dataclass_fields__:
    _cp = dataclasses.replace(_cp, use_tc_tiling_on_sc=False)

_i32 = jnp.int32


def _rnd128(v):
    return jnp.bitwise_and(v + 127, -128)


def _bin_edges(edge_index, edge_timestamp):
    @functools.partial(
        pl.kernel,
        out_type=(
            jax.ShapeDtypeStruct((NC * S,), _i32),
            jax.ShapeDtypeStruct((NC * S,), jnp.float32),
            jax.ShapeDtypeStruct((NC * S,), _i32),
            jax.ShapeDtypeStruct((GRID_PAD,), _i32),
        ),
        mesh=_mesh,
        compiler_params=_cp,
        scratch_types=[
            pltpu.VMEM((CH,), _i32),
            pltpu.VMEM((CH,), jnp.float32),
            pltpu.VMEM((CH,), _i32),
            pltpu.VMEM((CH,), _i32),
            pltpu.VMEM((128,), _i32),
            pltpu.VMEM((128,), _i32),
            pltpu.VMEM((128,), _i32),
            pltpu.VMEM((128,), _i32),
            pltpu.VMEM((NS, 128), _i32),
            pltpu.VMEM_SHARED((NS, 128), _i32),
            pltpu.SemaphoreType.DMA,
        ],
    )
    def k(ei, ts, bsrc, bts, bdst, counts,
          srcb, tsb, dstb, zb, cntb, curb, posb, cixb, gridb, shared, sem):
        c = lax.axis_index("c")
        s = lax.axis_index("s")
        bias_v, _ = plsc.scan_count(jnp.zeros((16,), _i32))
        bias = bias_v[0]

        @pl.loop(0, CH, step=16)
        def _(i):
            zb[pl.ds(i, 16)] = jnp.zeros((16,), _i32)

        @pl.loop(0, 128, step=16)
        def _(i):
            cntb[pl.ds(i, 16)] = jnp.zeros((16,), _i32)

        zbase = c * S + s * (ZCH * CH)

        @pl.loop(0, ZCH)
        def _(z):
            pltpu.sync_copy(zb, bsrc.at[pl.ds(zbase + z * CH, CH)])

        gstart = 781 * s + jnp.minimum(s, 4)
        base_e = c * EC + gstart * G

        def count_block(start, n):
            pltpu.sync_copy(ei.at[1, pl.ds(start, n)], dstb.at[pl.ds(0, n)])

            @pl.loop(0, n, step=16)
            def _(i):
                b16 = lax.shift_right_logical(dstb[pl.ds(i, 16)], 10)
                cnt, last = plsc.scan_count(b16)
                plsc.addupdate_scatter(cntb, [b16], cnt - bias + 1, mask=last)

        @pl.loop(0, 48)
        def _(ci):
            count_block(base_e + ci * CH, CH)

        @pl.when(s < 4)
        def _():
            count_block(base_e + 48 * CH, 1792)

        @pl.when(s >= 4)
        def _():
            count_block(base_e + 48 * CH, 1664)

        pltpu.sync_copy(cntb, shared.at[s])
        plsc.subcore_barrier()
        pltpu.sync_copy(shared, gridb)

        @pl.loop(0, 128, step=16)
        def _(i):
            fl = lax.iota(_i32, 16) + i
            cixb[pl.ds(i, 16)] = jnp.where(
                fl < NB, c * (NB * NS) + fl * NS + s, GRID + fl - NB)

        pltpu.sync_copy(cntb, counts.at[cixb])

        run = c * S
        for j in range(7):
            t_acc = jnp.zeros((16,), _i32)
            p_acc = jnp.zeros((16,), _i32)
            for sp in range(NS):
                r = _rnd128(gridb[sp, pl.ds(j * 16, 16)])
                t_acc = t_acc + r
                p_acc = p_acc + jnp.where(sp < s, r, 0)
            excl = plsc.cumsum(t_acc) - t_acc + run
            curb[pl.ds(j * 16, 16)] = excl + p_acc
            run = run + jnp.sum(t_acc)

        def scatter_block(start, n):
            pltpu.sync_copy(ei.at[0, pl.ds(start, n)], srcb.at[pl.ds(0, n)])
            pltpu.sync_copy(ts.at[pl.ds(start, n)], tsb.at[pl.ds(0, n)])
            pltpu.sync_copy(ei.at[1, pl.ds(start, n)], dstb.at[pl.ds(0, n)])
            @pl.loop(0, n // G)
            def _(g):
                gg = g * G
                for i in range(8):
                    b16 = lax.shift_right_logical(
                        dstb[pl.ds(gg + i * 16, 16)], 10)
                    cnt, last = plsc.scan_count(b16)
                    basev = plsc.load_gather(curb, [b16])
                    posb[pl.ds(i * 16, 16)] = basev + cnt - bias
                    plsc.addupdate_scatter(curb, [b16], cnt - bias + 1,
                                           mask=last)
                pltpu.sync_copy(srcb.at[pl.ds(gg, G)], bsrc.at[posb])
                pltpu.sync_copy(tsb.at[pl.ds(gg, G)], bts.at[posb])
                pltpu.sync_copy(dstb.at[pl.ds(gg, G)], bdst.at[posb])

        @pl.loop(0, 48)
        def _(ci):
            scatter_block(base_e + ci * CH, CH)

        @pl.when(s < 4)
        def _():
            scatter_block(base_e + 48 * CH, 1792)

        @pl.when(s >= 4)
        def _():
            scatter_block(base_e + 48 * CH, 1664)

    return k(edge_index, edge_timestamp)


def _aggregate(feat, bsrc, bts, bdst, counts):
    @functools.partial(
        pl.kernel,
        out_type=jax.ShapeDtypeStruct((NPAD * F,), jnp.float32),
        mesh=_mesh,
        compiler_params=_cp,
        scratch_types=[
            pltpu.VMEM((GRID_PAD,), _i32),
            pltpu.VMEM((GRID_PAD,), _i32),
            pltpu.VMEM(((K + 1) * F,), jnp.float32),
            pltpu.VMEM((G,), _i32),
            pltpu.VMEM((G, F), jnp.float32),
            pltpu.VMEM((G,), _i32),
            pltpu.VMEM((G,), jnp.float32),
            pltpu.SemaphoreType.DMA,
        ],
    )
    def k(ft, bs, bt, bd, cnts, agg,
          gridb, offsb, acc, ixb, rows, dstb, tsb, sem):
        c = lax.axis_index("c")
        s = lax.axis_index("s")
        wid = c * NS + s
        pltpu.sync_copy(cnts, gridb)

        run = jnp.zeros((), _i32)
        for j in range(GRID_PAD // 16):
            if j == (NB * NS) // 16:
                run = jnp.zeros((), _i32)
            v = gridb[pl.ds(j * 16, 16)]
            r = _rnd128(v)
            offsb[pl.ds(j * 16, 16)] = plsc.cumsum(r) - r + run
            run = run + jnp.sum(r)

        def scalar_read(ref, idx):
            return plsc.load_gather(ref, [jnp.full((16,), idx, _i32)])[0]

        @pl.loop(0, 4)
        def _(j2):
            bin_ = jnp.where(j2 < 3, 3 * wid + j2, 96 + wid)
            valid_bin = jnp.logical_or(j2 < 3, wid < NC)

            @pl.when(valid_bin)
            def _():
                @pl.loop(0, K * F, step=16)
                def _(i):
                    acc[pl.ds(i, 16)] = jnp.zeros((16,), jnp.float32)

                @pl.loop(0, NC * NS)
                def _(k2):
                    c2 = lax.shift_right_logical(k2, 4)
                    s2 = jnp.bitwise_and(k2, 15)
                    fl = c2 * (NB * NS) + bin_ * NS + s2
                    n = scalar_read(gridb, fl)
                    st = pl.multiple_of(
                        scalar_read(offsb, fl) + c2 * S, G)
                    ng = lax.shift_right_logical(n + G - 1, 7)

                    def grp(g, _):
                        gs = st + g * G
                        pltpu.sync_copy(bs.at[pl.ds(gs, G)], ixb)
                        pltpu.async_copy(ft.at[ixb], rows, sem).wait()
                        pltpu.sync_copy(bd.at[pl.ds(gs, G)], dstb)
                        pltpu.sync_copy(bt.at[pl.ds(gs, G)], tsb)
                        m = jnp.minimum(n - g * G, G)
                        nv = lax.shift_right_logical(m + 15, 4)

                        def vrg(i, _):
                            d16 = dstb[pl.ds(i * 16, 16)]
                            t16 = tsb[pl.ds(i * 16, 16)]
                            for lane in range(16):
                                valid = (i * 16 + lane) < m
                                off = jnp.where(
                                    valid, d16[lane] - bin_ * K, K) * F
                                row = rows[i * 16 + lane, :] * t16[lane]
                                cur = acc[pl.ds(off, 16)]
                                acc[pl.ds(off, 16)] = jnp.maximum(cur, row)
                            return 0

                        lax.fori_loop(0, nv, vrg, 0)
                        return 0

                    lax.fori_loop(0, ng, grp, 0)

                pltpu.sync_copy(acc.at[pl.ds(0, K * F)],
                                agg.at[pl.ds(bin_ * (K * F), K * F)])

    return k(feat, bsrc, bts, bdst, counts)


def _linear_block(a_ref, w_ref, b_ref, o_ref, *, relu):
    out = jnp.dot(a_ref[...], w_ref[...].T,
                  preferred_element_type=jnp.float32) + b_ref[...][None, :]
    if relu:
        out = jnp.maximum(out, 0.0)
    o_ref[...] = out


def _linear(a, w, b, relu):
    m, kdim = a.shape
    n = w.shape[0]
    bm = 3584
    return pl.pallas_call(
        lambda a_ref, w_ref, b_ref, o_ref: _linear_block(
            a_ref, w_ref, b_ref, o_ref, relu=relu),
        grid=(m // bm,),
        in_specs=[
            pl.BlockSpec((bm, kdim), lambda i: (i, 0)),
            pl.BlockSpec((n, kdim), lambda i: (0, 0)),
            pl.BlockSpec((n,), lambda i: (0,)),
        ],
        out_specs=pl.BlockSpec((bm, n), lambda i: (i, 0)),
        out_shape=jax.ShapeDtypeStruct((m, n), jnp.float32),
    )(a, w, b)


def kernel(x, edge_index, edge_timestamp, W1, b1, W2, b2):
    edge_index = edge_index.astype(jnp.int32)

    bsrc, bts, bdst, counts = _bin_edges(edge_index, edge_timestamp)

    agg1 = _aggregate(x, bsrc, bts, bdst, counts).reshape(NPAD, F)

    W1p = jnp.concatenate([W1, jnp.zeros((F - W1.shape[0], F), jnp.float32)], 0)
    b1p = jnp.concatenate([b1, jnp.zeros((F - b1.shape[0],), jnp.float32)], 0)
    h = _linear(agg1, W1p, b1p, relu=True)

    agg2 = _aggregate(h, bsrc, bts, bdst, counts).reshape(NPAD, F)

    W2p = jnp.concatenate(
        [W2, jnp.zeros((W2.shape[0], F - W2.shape[1]), jnp.float32)], 1)
    out = _linear(agg2, W2p, b2, relu=False)
    return out[:N]


# speedup vs baseline: 2.7067x; 1.0002x over previous
import dataclasses
import functools

import jax
import jax.numpy as jnp
from jax import lax
from jax.experimental import pallas as pl
from jax.experimental.pallas import tpu as pltpu
from jax.experimental.pallas import tpu_sc as plsc

N = 100_000
E = 3_200_000
F = 16
K = 1024
NB = 98
NPAD = NB * K
NC, NS = 2, 16
EC = E // NC
S = 1_802_240
G = 128
CH = 2048
GRID = NC * NB * NS
GRID_PAD = 3200
ZCH = 55

_mesh = plsc.VectorSubcoreMesh(core_axis_name="c", subcore_axis_name="s")
_cp = pltpu.CompilerParams()
if "needs_layout_passes" in pltpu.CompilerParams.__dataclass_fields__:
    _cp = dataclasses.replace(_cp, needs_layout_passes=False)
if "use_tc_tiling_on_sc" in pltpu.CompilerParams.__dataclass_fields__:
    _cp = dataclasses.replace(_cp, use_tc_tiling_on_sc=False)

_i32 = jnp.int32


def _rnd128(v):
    return jnp.bitwise_and(v + 127, -128)


def _bin_edges(edge_index, edge_timestamp):
    @functools.partial(
        pl.kernel,
        out_type=(
            jax.ShapeDtypeStruct((NC * S,), _i32),
            jax.ShapeDtypeStruct((NC * S,), jnp.float32),
            jax.ShapeDtypeStruct((NC * S,), _i32),
            jax.ShapeDtypeStruct((GRID_PAD,), _i32),
        ),
        mesh=_mesh,
        compiler_params=_cp,
        scratch_types=[
            pltpu.VMEM((CH,), _i32),
            pltpu.VMEM((CH,), jnp.float32),
            pltpu.VMEM((CH,), _i32),
            pltpu.VMEM((CH,), _i32),
            pltpu.VMEM((128,), _i32),
            pltpu.VMEM((128,), _i32),
            pltpu.VMEM((128,), _i32),
            pltpu.VMEM((128,), _i32),
            pltpu.VMEM((128,), _i32),
            pltpu.VMEM((NS, 128), _i32),
            pltpu.VMEM_SHARED((NS, 128), _i32),
            pltpu.SemaphoreType.DMA,
        ],
    )
    def k(ei, ts, bsrc, bts, bdst, counts,
          srcb, tsb, dstb, zb, cntb, curb, posb, posb2, cixb, gridb, shared,
          sem):
        c = lax.axis_index("c")
        s = lax.axis_index("s")
        bias_v, _ = plsc.scan_count(jnp.zeros((16,), _i32))
        bias = bias_v[0]

        @pl.loop(0, CH, step=16)
        def _(i):
            zb[pl.ds(i, 16)] = jnp.zeros((16,), _i32)

        @pl.loop(0, 128, step=16)
        def _(i):
            cntb[pl.ds(i, 16)] = jnp.zeros((16,), _i32)

        zbase = c * S + s * (ZCH * CH)

        @pl.loop(0, ZCH)
        def _(z):
            pltpu.sync_copy(zb, bsrc.at[pl.ds(zbase + z * CH, CH)])

        gstart = 781 * s + jnp.minimum(s, 4)
        base_e = c * EC + gstart * G

        def count_block(start, n):
            pltpu.sync_copy(ei.at[1, pl.ds(start, n)], dstb.at[pl.ds(0, n)])

            @pl.loop(0, n, step=16)
            def _(i):
                b16 = lax.shift_right_logical(dstb[pl.ds(i, 16)], 10)
                cnt, last = plsc.scan_count(b16)
                plsc.addupdate_scatter(cntb, [b16], cnt - bias + 1, mask=last)

        @pl.loop(0, 48)
        def _(ci):
            count_block(base_e + ci * CH, CH)

        @pl.when(s < 4)
        def _():
            count_block(base_e + 48 * CH, 1792)

        @pl.when(s >= 4)
        def _():
            count_block(base_e + 48 * CH, 1664)

        pltpu.sync_copy(cntb, shared.at[s])
        plsc.subcore_barrier()
        pltpu.sync_copy(shared, gridb)

        @pl.loop(0, 128, step=16)
        def _(i):
            fl = lax.iota(_i32, 16) + i
            cixb[pl.ds(i, 16)] = jnp.where(
                fl < NB, c * (NB * NS) + fl * NS + s, GRID + fl - NB)

        pltpu.sync_copy(cntb, counts.at[cixb])

        run = c * S
        for j in range(7):
            t_acc = jnp.zeros((16,), _i32)
            p_acc = jnp.zeros((16,), _i32)
            for sp in range(NS):
                r = _rnd128(gridb[sp, pl.ds(j * 16, 16)])
                t_acc = t_acc + r
                p_acc = p_acc + jnp.where(sp < s, r, 0)
            excl = plsc.cumsum(t_acc) - t_acc + run
            curb[pl.ds(j * 16, 16)] = excl + p_acc
            run = run + jnp.sum(t_acc)

        def _wait3():
            pltpu.make_async_copy(srcb.at[pl.ds(0, G)], bsrc.at[posb],
                                  sem).wait()
            pltpu.make_async_copy(tsb.at[pl.ds(0, G)], bts.at[posb],
                                  sem).wait()
            pltpu.make_async_copy(dstb.at[pl.ds(0, G)], bdst.at[posb],
                                  sem).wait()

        def scatter_block(start, n):
            pltpu.sync_copy(ei.at[0, pl.ds(start, n)], srcb.at[pl.ds(0, n)])
            pltpu.sync_copy(ts.at[pl.ds(start, n)], tsb.at[pl.ds(0, n)])
            pltpu.sync_copy(ei.at[1, pl.ds(start, n)], dstb.at[pl.ds(0, n)])
            for g in range(n // G):
                pb = posb if g % 2 == 0 else posb2
                gg = g * G
                for i in range(8):
                    b16 = lax.shift_right_logical(
                        dstb[pl.ds(gg + i * 16, 16)], 10)
                    cnt, last = plsc.scan_count(b16)
                    basev = plsc.load_gather(curb, [b16])
                    pb[pl.ds(i * 16, 16)] = basev + cnt - bias
                    plsc.addupdate_scatter(curb, [b16], cnt - bias + 1,
                                           mask=last)
                pltpu.async_copy(srcb.at[pl.ds(gg, G)], bsrc.at[pb], sem)
                pltpu.async_copy(tsb.at[pl.ds(gg, G)], bts.at[pb], sem)
                pltpu.async_copy(dstb.at[pl.ds(gg, G)], bdst.at[pb], sem)
                if g >= 1:
                    _wait3()
            _wait3()

        @pl.loop(0, 48)
        def _(ci):
            scatter_block(base_e + ci * CH, CH)

        @pl.when(s < 4)
        def _():
            scatter_block(base_e + 48 * CH, 1792)

        @pl.when(s >= 4)
        def _():
            scatter_block(base_e + 48 * CH, 1664)

    return k(edge_index, edge_timestamp)


def _aggregate(feat, bsrc, bts, bdst, counts):
    @functools.partial(
        pl.kernel,
        out_type=jax.ShapeDtypeStruct((NPAD * F,), jnp.float32),
        mesh=_mesh,
        compiler_params=_cp,
        scratch_types=[
            pltpu.VMEM((GRID_PAD,), _i32),
            pltpu.VMEM((GRID_PAD,), _i32),
            pltpu.VMEM(((K + 1) * F,), jnp.float32),
            pltpu.VMEM((G,), _i32),
            pltpu.VMEM((G, F), jnp.float32),
            pltpu.VMEM((G,), _i32),
            pltpu.VMEM((G,), jnp.float32),
            pltpu.SemaphoreType.DMA,
        ],
    )
    def k(ft, bs, bt, bd, cnts, agg,
          gridb, offsb, acc, ixb, rows, dstb, tsb, sem):
        c = lax.axis_index("c")
        s = lax.axis_index("s")
        wid = c * NS + s
        pltpu.sync_copy(cnts, gridb)

        run = jnp.zeros((), _i32)
        for j in range(GRID_PAD // 16):
            if j == (NB * NS) // 16:
                run = jnp.zeros((), _i32)
            v = gridb[pl.ds(j * 16, 16)]
            r = _rnd128(v)
            offsb[pl.ds(j * 16, 16)] = plsc.cumsum(r) - r + run
            run = run + jnp.sum(r)

        def scalar_read(ref, idx):
            return plsc.load_gather(ref, [jnp.full((16,), idx, _i32)])[0]

        @pl.loop(0, 4)
        def _(j2):
            bin_ = jnp.where(j2 < 3, 3 * wid + j2, 96 + wid)
            valid_bin = jnp.logical_or(j2 < 3, wid < NC)

            @pl.when(valid_bin)
            def _():
                @pl.loop(0, K * F, step=16)
                def _(i):
                    acc[pl.ds(i, 16)] = jnp.zeros((16,), jnp.float32)

                @pl.loop(0, NC * NS)
                def _(k2):
                    c2 = lax.shift_right_logical(k2, 4)
                    s2 = jnp.bitwise_and(k2, 15)
                    fl = c2 * (NB * NS) + bin_ * NS + s2
                    n = scalar_read(gridb, fl)
                    st = pl.multiple_of(
                        scalar_read(offsb, fl) + c2 * S, G)
                    ng = lax.shift_right_logical(n + G - 1, 7)

                    def grp(g, _):
                        gs = st + g * G
                        pltpu.sync_copy(bs.at[pl.ds(gs, G)], ixb)
                        pltpu.async_copy(ft.at[ixb], rows, sem).wait()
                        pltpu.sync_copy(bd.at[pl.ds(gs, G)], dstb)
                        pltpu.sync_copy(bt.at[pl.ds(gs, G)], tsb)
                        m = jnp.minimum(n - g * G, G)
                        nv = lax.shift_right_logical(m + 15, 4)

                        def vrg(i, _):
                            d16 = dstb[pl.ds(i * 16, 16)]
                            t16 = tsb[pl.ds(i * 16, 16)]
                            for lane in range(16):
                                valid = (i * 16 + lane) < m
                                off = jnp.where(
                                    valid, d16[lane] - bin_ * K, K) * F
                                row = rows[i * 16 + lane, :] * t16[lane]
                                cur = acc[pl.ds(off, 16)]
                                acc[pl.ds(off, 16)] = jnp.maximum(cur, row)
                            return 0

                        lax.fori_loop(0, nv, vrg, 0)
                        return 0

                    lax.fori_loop(0, ng, grp, 0)

                pltpu.sync_copy(acc.at[pl.ds(0, K * F)],
                                agg.at[pl.ds(bin_ * (K * F), K * F)])

    return k(feat, bsrc, bts, bdst, counts)


def _linear_block(a_ref, w_ref, b_ref, o_ref, *, relu):
    out = jnp.dot(a_ref[...], w_ref[...].T,
                  preferred_element_type=jnp.float32) + b_ref[...][None, :]
    if relu:
        out = jnp.maximum(out, 0.0)
    o_ref[...] = out


def _linear(a, w, b, relu):
    m, kdim = a.shape
    n = w.shape[0]
    bm = 3584
    return pl.pallas_call(
        lambda a_ref, w_ref, b_ref, o_ref: _linear_block(
            a_ref, w_ref, b_ref, o_ref, relu=relu),
        grid=(m // bm,),
        in_specs=[
            pl.BlockSpec((bm, kdim), lambda i: (i, 0)),
            pl.BlockSpec((n, kdim), lambda i: (0, 0)),
            pl.BlockSpec((n,), lambda i: (0,)),
        ],
        out_specs=pl.BlockSpec((bm, n), lambda i: (i, 0)),
        out_shape=jax.ShapeDtypeStruct((m, n), jnp.float32),
    )(a, w, b)


def kernel(x, edge_index, edge_timestamp, W1, b1, W2, b2):
    edge_index = edge_index.astype(jnp.int32)

    bsrc, bts, bdst, counts = _bin_edges(edge_index, edge_timestamp)

    agg1 = _aggregate(x, bsrc, bts, bdst, counts).reshape(NPAD, F)

    W1p = jnp.concatenate([W1, jnp.zeros((F - W1.shape[0], F), jnp.float32)], 0)
    b1p = jnp.concatenate([b1, jnp.zeros((F - b1.shape[0],), jnp.float32)], 0)
    h = _linear(agg1, W1p, b1p, relu=True)

    agg2 = _aggregate(h, bsrc, bts, bdst, counts).reshape(NPAD, F)

    W2p = jnp.concatenate(
        [W2, jnp.zeros((W2.shape[0], F - W2.shape[1]), jnp.float32)], 1)
    out = _linear(agg2, W2p, b2, relu=False)
    return out[:N]
